# baseline (speedup 1.0000x reference)
import jax
import jax.numpy as jnp
from jax.experimental import pallas as pl
from jax.experimental.pallas import tpu as pltpu


_CHUNKS = 5


def _dma_tile4_kernel(x_hbm, o_hbm, vbuf, in_sems, out_sems):
    n, d = vbuf.shape
    h = n // _CHUNKS
    in_cps = []
    for c in range(_CHUNKS):
        rows = pl.ds(c * h, h)
        cp = pltpu.make_async_copy(
            x_hbm.at[rows, :], vbuf.at[rows, :], in_sems.at[c])
        cp.start()
        in_cps.append(cp)
    out_cps = []
    for c in range(_CHUNKS):
        in_cps[c].wait()
        rows = pl.ds(c * h, h)
        for j in range(4):
            cp = pltpu.make_async_copy(
                vbuf.at[rows, :], o_hbm.at[rows, pl.ds(j * d, d)],
                out_sems.at[c, j])
            cp.start()
            out_cps.append(cp)
    for cp in out_cps:
        cp.wait()


def kernel(x, edge_index):
    del edge_index
    n, d = x.shape
    out = pl.pallas_call(
        _dma_tile4_kernel,
        in_specs=[pl.BlockSpec(memory_space=pl.ANY)],
        out_specs=pl.BlockSpec(memory_space=pl.ANY),
        out_shape=jax.ShapeDtypeStruct((n, 4 * d), x.dtype),
        scratch_shapes=[
            pltpu.VMEM((n, d), x.dtype),
            pltpu.SemaphoreType.DMA((_CHUNKS,)),
            pltpu.SemaphoreType.DMA((_CHUNKS, 4)),
        ],
    )(x)
    return out


# speedup vs baseline: 1.0700x; 1.0700x over previous
import jax
import jax.numpy as jnp
from jax.experimental import pallas as pl
from jax.experimental.pallas import tpu as pltpu


_CHUNKS = 2


def _dma_tile4_kernel(x_hbm, o_hbm, vbuf, in_sems, out_sems):
    n, d = vbuf.shape
    h = n // _CHUNKS
    in_cps = []
    for c in range(_CHUNKS):
        rows = pl.ds(c * h, h)
        cp = pltpu.make_async_copy(
            x_hbm.at[rows, :], vbuf.at[rows, :], in_sems.at[c])
        cp.start()
        in_cps.append(cp)
    out_cps = []
    for c in range(_CHUNKS):
        in_cps[c].wait()
        rows = pl.ds(c * h, h)
        for j in range(4):
            cp = pltpu.make_async_copy(
                vbuf.at[rows, :], o_hbm.at[rows, pl.ds(j * d, d)],
                out_sems.at[c, j])
            cp.start()
            out_cps.append(cp)
    for cp in out_cps:
        cp.wait()


def kernel(x, edge_index):
    del edge_index
    n, d = x.shape
    out = pl.pallas_call(
        _dma_tile4_kernel,
        in_specs=[pl.BlockSpec(memory_space=pl.ANY)],
        out_specs=pl.BlockSpec(memory_space=pl.ANY),
        out_shape=jax.ShapeDtypeStruct((n, 4 * d), x.dtype),
        scratch_shapes=[
            pltpu.VMEM((n, d), x.dtype),
            pltpu.SemaphoreType.DMA((_CHUNKS,)),
            pltpu.SemaphoreType.DMA((_CHUNKS, 4)),
        ],
    )(x)
    return out
